# Initial kernel scaffold; baseline (speedup 1.0000x reference)
import functools

import jax
import jax.numpy as jnp
from jax import lax
from jax.experimental import pallas as pl
from jax.experimental.pallas import tpu as pltpu
from jax.experimental.pallas import tpu_sc as plsc

_NC = 2
_NS = 16
_L = 16

_CH = 80


def _sc_scatter_body(edge_hbm, idx_hbm, sums_out, cnts_out, buf, idxbuf,
                     onesb, zc, acc_sums, acc_cnts):
    n_nodes = acc_sums.shape[0]
    n_edges = edge_hbm.shape[0]
    c = lax.axis_index("c")
    s = lax.axis_index("s")

    zero = jnp.zeros((_L,), jnp.float32)
    one = jnp.ones((_L,), jnp.float32)

    def _fill_row(r, _):
        for j in range(128 // _L):
            buf[0, r, pl.ds(j * _L, _L)] = zero
        zc[r] = zero
        onesb[r] = one
        return 0

    lax.fori_loop(0, _CH, _fill_row, 0)

    rows_pt = n_nodes // _NS
    row0 = s * rows_pt
    n_full = rows_pt // _CH
    rem = rows_pt - n_full * _CH
    for j in range(n_full):
        pltpu.sync_copy(buf.at[0], acc_sums.at[pl.ds(row0 + j * _CH, _CH)])
        pltpu.sync_copy(zc, acc_cnts.at[pl.ds(row0 + j * _CH, _CH)])
    if rem:
        pltpu.sync_copy(buf.at[0, pl.ds(0, rem)],
                        acc_sums.at[pl.ds(row0 + n_full * _CH, rem)])
        pltpu.sync_copy(zc.at[pl.ds(0, rem)],
                        acc_cnts.at[pl.ds(row0 + n_full * _CH, rem)])
    plsc.subcore_barrier()

    edges_pw = n_edges // (_NC * _NS)
    wbase = (c * _NS + s) * edges_pw
    n_iter = edges_pw // _CH

    def _it(k, _):
        base = wbase + k * _CH
        pltpu.sync_copy(edge_hbm.at[pl.ds(base, _CH)], buf.at[0])
        pltpu.sync_copy(idx_hbm.at[pl.ds(base, _CH)], idxbuf.at[0])
        pltpu.sync_copy(buf.at[0], acc_sums.at[idxbuf.at[0]], add=True)
        pltpu.sync_copy(onesb, acc_cnts.at[idxbuf.at[0]], add=True)
        return 0

    lax.fori_loop(0, n_iter, _it, 0)
    plsc.subcore_barrier()

    pltpu.sync_copy(acc_sums.at[pl.ds(row0, rows_pt)],
                    sums_out.at[c, pl.ds(row0, rows_pt)])
    pltpu.sync_copy(acc_cnts.at[pl.ds(row0, rows_pt)],
                    cnts_out.at[c, pl.ds(row0, rows_pt)])


def _make_sc_scatter(n_nodes, n_edges, dim):
    mesh = plsc.VectorSubcoreMesh(core_axis_name="c", subcore_axis_name="s")

    @functools.partial(
        pl.kernel,
        out_type=(
            jax.ShapeDtypeStruct((_NC, n_nodes, dim), jnp.float32),
            jax.ShapeDtypeStruct((_NC, n_nodes, _L), jnp.float32),
        ),
        mesh=mesh,
        scratch_types=[
            pltpu.VMEM((2, _CH, dim), jnp.float32),
            pltpu.VMEM((2, _CH), jnp.int32),
            pltpu.VMEM((_CH, _L), jnp.float32),
            pltpu.VMEM((_CH, _L), jnp.float32),
            pltpu.VMEM_SHARED((n_nodes, dim), jnp.float32),
            pltpu.VMEM_SHARED((n_nodes, _L), jnp.float32),
        ],
    )
    def sc_scatter(edge_hbm, idx_hbm, sums_out, cnts_out, buf, idxbuf,
                   onesb, zc, acc_sums, acc_cnts):
        _sc_scatter_body(edge_hbm, idx_hbm, sums_out, cnts_out, buf, idxbuf,
                         onesb, zc, acc_sums, acc_cnts)

    return sc_scatter


def _tc_finish_body(sums_ref, cnts_ref, x_ref, batch_ref, u_ref, W0_ref,
                    b0_ref, W1_ref, b1_ref, W2_ref, b2_ref, out_ref):
    n_nodes = x_ref.shape[0]
    n_graphs = u_ref.shape[0]
    s = sums_ref[0] + sums_ref[1]
    cnt = jnp.sum(cnts_ref[0] + cnts_ref[1], axis=1, keepdims=True) * (1.0 / _L)
    ue_node = s / jnp.maximum(cnt, 1.0)
    bvec = batch_ref[:]
    giota = lax.broadcasted_iota(jnp.int32, (n_graphs, n_nodes), 0)
    onehot = (giota == bvec).astype(jnp.float32)
    acc_ue = jnp.dot(onehot, ue_node, preferred_element_type=jnp.float32)
    acc_uv = jnp.dot(onehot, x_ref[:], preferred_element_type=jnp.float32)
    npg = jnp.sum(onehot, axis=1, keepdims=True)
    inv = 1.0 / jnp.maximum(npg, 1.0)
    comb = jnp.concatenate([acc_ue * inv, acc_uv * inv, u_ref[:]], axis=1)
    h = jnp.maximum(
        jnp.dot(comb, W0_ref[:], preferred_element_type=jnp.float32)
        + b0_ref[:], 0.0)
    h = jnp.maximum(
        jnp.dot(h, W1_ref[:], preferred_element_type=jnp.float32)
        + b1_ref[:], 0.0)
    out_ref[:] = jnp.maximum(
        jnp.dot(h, W2_ref[:], preferred_element_type=jnp.float32)
        + b2_ref[:], 0.0)


def kernel(x, edge_index, edge_attr, u, batch, W0, b0, W1, b1, W2, b2):
    n_nodes, dim = x.shape
    n_edges = edge_attr.shape[0]
    n_graphs = u.shape[0]

    src = edge_index[0]
    sums, cnts = _make_sc_scatter(n_nodes, n_edges, dim)(edge_attr, src)

    out = pl.pallas_call(
        _tc_finish_body,
        out_shape=jax.ShapeDtypeStruct((n_graphs, dim), jnp.float32),
    )(sums, cnts, x, batch.reshape(1, n_nodes), u, W0, b0, W1, b1, W2, b2)
    return out


# speedup vs baseline: 4.1518x; 4.1518x over previous
import functools

import jax
import jax.numpy as jnp
from jax import lax
from jax.experimental import pallas as pl
from jax.experimental.pallas import tpu as pltpu
from jax.experimental.pallas import tpu_sc as plsc

_NC = 2
_NS = 16
_L = 16

_CH = 80


def _zero_fill_rows(ref, width):
    zero = jnp.zeros((_L,), jnp.float32)

    def _row(r, _):
        for j in range(width // _L):
            ref[r, pl.ds(j * _L, _L)] = zero
        return 0

    lax.fori_loop(0, ref.shape[0], _row, 0)


def _tile_row_split(n_nodes, s):
    rows_pt = (n_nodes // _NS) // 8 * 8
    rem_rows = n_nodes - _NS * rows_pt
    return rows_pt, rem_rows, s * rows_pt


def _zero_acc_slice(zsrc, acc, row0, rows_pt, rem_rows, s):
    n_full = rows_pt // _CH
    rem = rows_pt - n_full * _CH
    for j in range(n_full):
        pltpu.sync_copy(zsrc, acc.at[pl.ds(row0 + j * _CH, _CH)])
    if rem:
        pltpu.sync_copy(zsrc.at[pl.ds(0, rem)],
                        acc.at[pl.ds(row0 + n_full * _CH, rem)])

    @pl.when(s == _NS - 1)
    def _tail():
        pltpu.sync_copy(zsrc.at[pl.ds(0, rem_rows)],
                        acc.at[pl.ds(_NS * rows_pt, rem_rows)])


def _copy_acc_out(acc, out, c, row0, rows_pt, rem_rows, s):
    pltpu.sync_copy(acc.at[pl.ds(row0, rows_pt)],
                    out.at[c, pl.ds(row0, rows_pt)])

    @pl.when(s == _NS - 1)
    def _tail():
        pltpu.sync_copy(acc.at[pl.ds(_NS * rows_pt, rem_rows)],
                        out.at[c, pl.ds(_NS * rows_pt, rem_rows)])


def _make_sc_sum(n_nodes, n_edges, dim):
    mesh = plsc.VectorSubcoreMesh(core_axis_name="c", subcore_axis_name="s")

    @functools.partial(
        pl.kernel,
        out_type=jax.ShapeDtypeStruct((_NC, n_nodes, dim), jnp.float32),
        mesh=mesh,
        scratch_types=[
            pltpu.VMEM((2, _CH, dim), jnp.float32),
            pltpu.VMEM((2, _CH), jnp.int32),
            pltpu.VMEM_SHARED((n_nodes, dim), jnp.float32),
        ],
    )
    def sc_sum(edge_hbm, idx_hbm, sums_out, buf, idxbuf, acc):
        c = lax.axis_index("c")
        s = lax.axis_index("s")
        _zero_fill_rows(buf.at[0], dim)
        rows_pt, rem_rows, row0 = _tile_row_split(n_nodes, s)
        _zero_acc_slice(buf.at[0], acc, row0, rows_pt, rem_rows, s)
        plsc.subcore_barrier()

        edges_pw = n_edges // (_NC * _NS)
        wbase = (c * _NS + s) * edges_pw
        n_iter = edges_pw // _CH

        def _it(k, _):
            base = wbase + k * _CH
            pltpu.sync_copy(edge_hbm.at[pl.ds(base, _CH)], buf.at[0])
            pltpu.sync_copy(idx_hbm.at[pl.ds(base, _CH)], idxbuf.at[0])
            pltpu.sync_copy(buf.at[0], acc.at[idxbuf.at[0]], add=True)
            return 0

        lax.fori_loop(0, n_iter, _it, 0)
        plsc.subcore_barrier()
        _copy_acc_out(acc, sums_out, c, row0, rows_pt, rem_rows, s)

    return sc_sum


def _make_sc_count(n_nodes, n_edges):
    mesh = plsc.VectorSubcoreMesh(core_axis_name="c", subcore_axis_name="s")

    @functools.partial(
        pl.kernel,
        out_type=jax.ShapeDtypeStruct((_NC, n_nodes, _L), jnp.float32),
        mesh=mesh,
        scratch_types=[
            pltpu.VMEM((_CH, _L), jnp.float32),
            pltpu.VMEM((_CH, _L), jnp.float32),
            pltpu.VMEM((2, _CH), jnp.int32),
            pltpu.VMEM_SHARED((n_nodes, _L), jnp.float32),
        ],
        compiler_params=pltpu.CompilerParams(use_tc_tiling_on_sc=False),
    )
    def sc_count(idx_hbm, cnts_out, onesb, zc, idxbuf, acc):
        c = lax.axis_index("c")
        s = lax.axis_index("s")
        one = jnp.ones((_L,), jnp.float32)

        def _row(r, _):
            onesb[r] = one
            return 0

        lax.fori_loop(0, _CH, _row, 0)
        _zero_fill_rows(zc, _L)
        rows_pt, rem_rows, row0 = _tile_row_split(n_nodes, s)
        _zero_acc_slice(zc, acc, row0, rows_pt, rem_rows, s)
        plsc.subcore_barrier()

        edges_pw = n_edges // (_NC * _NS)
        wbase = (c * _NS + s) * edges_pw
        n_iter = edges_pw // _CH

        def _it(k, _):
            base = wbase + k * _CH
            pltpu.sync_copy(idx_hbm.at[pl.ds(base, _CH)], idxbuf.at[0])
            pltpu.sync_copy(onesb, acc.at[idxbuf.at[0]], add=True)
            return 0

        lax.fori_loop(0, n_iter, _it, 0)
        plsc.subcore_barrier()
        _copy_acc_out(acc, cnts_out, c, row0, rows_pt, rem_rows, s)

    return sc_count


def _tc_finish_body(sums_ref, cnts_ref, x_ref, batch_ref, u_ref, W0_ref,
                    b0_ref, W1_ref, b1_ref, W2_ref, b2_ref, out_ref):
    n_nodes = x_ref.shape[0]
    n_graphs = u_ref.shape[0]
    s = sums_ref[0] + sums_ref[1]
    cnt = jnp.sum(cnts_ref[0] + cnts_ref[1], axis=1, keepdims=True) * (1.0 / _L)
    ue_node = s / jnp.maximum(cnt, 1.0)
    bvec = batch_ref[:]
    giota = lax.broadcasted_iota(jnp.int32, (n_graphs, n_nodes), 0)
    onehot = (giota == bvec).astype(jnp.float32)
    acc_ue = jnp.dot(onehot, ue_node, preferred_element_type=jnp.float32)
    acc_uv = jnp.dot(onehot, x_ref[:], preferred_element_type=jnp.float32)
    npg = jnp.sum(onehot, axis=1, keepdims=True)
    inv = 1.0 / jnp.maximum(npg, 1.0)
    comb = jnp.concatenate([acc_ue * inv, acc_uv * inv, u_ref[:]], axis=1)
    h = jnp.maximum(
        jnp.dot(comb, W0_ref[:], preferred_element_type=jnp.float32)
        + b0_ref[:], 0.0)
    h = jnp.maximum(
        jnp.dot(h, W1_ref[:], preferred_element_type=jnp.float32)
        + b1_ref[:], 0.0)
    out_ref[:] = jnp.maximum(
        jnp.dot(h, W2_ref[:], preferred_element_type=jnp.float32)
        + b2_ref[:], 0.0)


def kernel(x, edge_index, edge_attr, u, batch, W0, b0, W1, b1, W2, b2):
    n_nodes, dim = x.shape
    n_edges = edge_attr.shape[0]
    n_graphs = u.shape[0]

    src = edge_index[0]
    sums = _make_sc_sum(n_nodes, n_edges, dim)(edge_attr, src)
    cnts = _make_sc_count(n_nodes, n_edges)(src)

    out = pl.pallas_call(
        _tc_finish_body,
        out_shape=jax.ShapeDtypeStruct((n_graphs, dim), jnp.float32),
    )(sums, cnts, x, batch.reshape(1, n_nodes), u, W0, b0, W1, b1, W2, b2)
    return out


# speedup vs baseline: 8.9723x; 2.1611x over previous
import functools

import jax
import jax.numpy as jnp
from jax import lax
from jax.experimental import pallas as pl
from jax.experimental.pallas import tpu as pltpu
from jax.experimental.pallas import tpu_sc as plsc

_NC = 2
_NS = 16
_L = 16

_SCH = 80
_NSUB = 1
_GCH = _SCH * _NSUB


def _zero_fill_rows(ref, width):
    zero = jnp.zeros((_L,), jnp.float32)

    def _row(r, _):
        for j in range(width // _L):
            ref[r, pl.ds(j * _L, _L)] = zero
        return 0

    lax.fori_loop(0, ref.shape[0], _row, 0)


def _make_sc_scatter(n_nodes, n_edges, dim):
    mesh = plsc.VectorSubcoreMesh(core_axis_name="c", subcore_axis_name="s")

    @functools.partial(
        pl.kernel,
        out_type=(
            jax.ShapeDtypeStruct((_NC, n_nodes, dim), jnp.float32),
            jax.ShapeDtypeStruct((_NC, n_nodes, _L), jnp.float32),
        ),
        mesh=mesh,
        scratch_types=[
            pltpu.VMEM((2, _GCH, dim), jnp.float32),
            pltpu.VMEM((2, _NSUB, _SCH), jnp.int32),
            pltpu.VMEM((_SCH, _L), jnp.float32),
            pltpu.VMEM((_SCH, _L), jnp.float32),
            pltpu.VMEM_SHARED((n_nodes, dim), jnp.float32),
            pltpu.VMEM_SHARED((n_nodes, _L), jnp.float32),
            pltpu.SemaphoreType.DMA,
            pltpu.SemaphoreType.DMA,
            pltpu.SemaphoreType.DMA,
            pltpu.SemaphoreType.DMA,
        ],
        compiler_params=pltpu.CompilerParams(use_tc_tiling_on_sc=False),
    )
    def sc_scatter(edge_hbm, idx_hbm, sums_out, cnts_out, buf, idxbuf, onesb,
                   zc, acc, cacc, esem0, esem1, isem0, isem1):
        c = lax.axis_index("c")
        s = lax.axis_index("s")
        esem = (esem0, esem1)
        isem = (isem0, isem1)

        one = jnp.ones((_L,), jnp.float32)

        def _ones_row(r, _):
            onesb[r] = one
            return 0

        lax.fori_loop(0, _SCH, _ones_row, 0)
        _zero_fill_rows(zc, _L)
        _zero_fill_rows(buf.at[0], dim)

        rows_pt = (n_nodes // _NS) // 8 * 8
        rem_rows = n_nodes - _NS * rows_pt
        row0 = s * rows_pt
        n_zfull = rows_pt // _GCH
        for j in range(n_zfull):
            pltpu.sync_copy(buf.at[0, pl.ds(0, _GCH)],
                            acc.at[pl.ds(row0 + j * _GCH, _GCH)])
        zrem_a = rows_pt - n_zfull * _GCH
        if zrem_a:
            pltpu.sync_copy(buf.at[0, pl.ds(0, zrem_a)],
                            acc.at[pl.ds(row0 + n_zfull * _GCH, zrem_a)])
        n_zfull_c = rows_pt // _SCH
        for j in range(n_zfull_c):
            pltpu.sync_copy(zc, cacc.at[pl.ds(row0 + j * _SCH, _SCH)])
        zrem = rows_pt - n_zfull_c * _SCH
        if zrem:
            pltpu.sync_copy(zc.at[pl.ds(0, zrem)],
                            cacc.at[pl.ds(row0 + n_zfull_c * _SCH, zrem)])

        @pl.when(s == _NS - 1)
        def _zero_tail():
            pltpu.sync_copy(buf.at[0, pl.ds(0, rem_rows)],
                            acc.at[pl.ds(_NS * rows_pt, rem_rows)])
            pltpu.sync_copy(zc.at[pl.ds(0, rem_rows)],
                            cacc.at[pl.ds(_NS * rows_pt, rem_rows)])

        plsc.subcore_barrier()

        edges_pw = n_edges // (_NC * _NS)
        wbase = (c * _NS + s) * edges_pw
        n_iter = edges_pw // _GCH

        def _start_gather(i, b):
            pltpu.async_copy(edge_hbm.at[pl.ds(wbase + i * _GCH, _GCH)],
                             buf.at[b], esem[b])
            pltpu.async_copy(
                idx_hbm.at[pl.ds((wbase + i * _GCH) // _SCH, _NSUB)],
                idxbuf.at[b], isem[b])

        def _wait_gather(b):
            pltpu.make_async_copy(edge_hbm.at[pl.ds(0, _GCH)], buf.at[b],
                                  esem[b]).wait()
            pltpu.make_async_copy(idx_hbm.at[pl.ds(0, _NSUB)], idxbuf.at[b],
                                  isem[b]).wait()

        def _consume(i, b):
            _wait_gather(b)
            for j in range(_NSUB):
                pltpu.sync_copy(buf.at[b, pl.ds(j * _SCH, _SCH)],
                                acc.at[idxbuf.at[b, j]], add=True)
                pltpu.sync_copy(onesb, cacc.at[idxbuf.at[b, j]], add=True)

            @pl.when(i + 2 < n_iter)
            def _next():
                _start_gather(i + 2, b)

        _start_gather(0, 0)
        _start_gather(1, 1)

        def _outer(g, _):
            _consume(2 * g, 0)
            _consume(2 * g + 1, 1)
            return 0

        lax.fori_loop(0, n_iter // 2, _outer, 0)
        if n_iter % 2:
            _consume(n_iter - 1, 0)
        plsc.subcore_barrier()

        pltpu.sync_copy(acc.at[pl.ds(row0, rows_pt)],
                        sums_out.at[c, pl.ds(row0, rows_pt)])
        pltpu.sync_copy(cacc.at[pl.ds(row0, rows_pt)],
                        cnts_out.at[c, pl.ds(row0, rows_pt)])

        @pl.when(s == _NS - 1)
        def _copy_tail():
            pltpu.sync_copy(acc.at[pl.ds(_NS * rows_pt, rem_rows)],
                            sums_out.at[c, pl.ds(_NS * rows_pt, rem_rows)])
            pltpu.sync_copy(cacc.at[pl.ds(_NS * rows_pt, rem_rows)],
                            cnts_out.at[c, pl.ds(_NS * rows_pt, rem_rows)])

    return sc_scatter


def _tc_finish_body(sums_ref, cnts_ref, x_ref, batch_ref, u_ref, W0_ref,
                    b0_ref, W1_ref, b1_ref, W2_ref, b2_ref, out_ref):
    n_nodes = x_ref.shape[0]
    n_graphs = u_ref.shape[0]
    s = sums_ref[0] + sums_ref[1]
    cnt = jnp.sum(cnts_ref[0] + cnts_ref[1], axis=1, keepdims=True) * (1.0 / _L)
    ue_node = s / jnp.maximum(cnt, 1.0)
    bvec = batch_ref[:]
    giota = lax.broadcasted_iota(jnp.int32, (n_graphs, n_nodes), 0)
    onehot = (giota == bvec).astype(jnp.float32)
    acc_ue = jnp.dot(onehot, ue_node, preferred_element_type=jnp.float32)
    acc_uv = jnp.dot(onehot, x_ref[:], preferred_element_type=jnp.float32)
    npg = jnp.sum(onehot, axis=1, keepdims=True)
    inv = 1.0 / jnp.maximum(npg, 1.0)
    comb = jnp.concatenate([acc_ue * inv, acc_uv * inv, u_ref[:]], axis=1)
    h = jnp.maximum(
        jnp.dot(comb, W0_ref[:], preferred_element_type=jnp.float32)
        + b0_ref[:], 0.0)
    h = jnp.maximum(
        jnp.dot(h, W1_ref[:], preferred_element_type=jnp.float32)
        + b1_ref[:], 0.0)
    out_ref[:] = jnp.maximum(
        jnp.dot(h, W2_ref[:], preferred_element_type=jnp.float32)
        + b2_ref[:], 0.0)


def kernel(x, edge_index, edge_attr, u, batch, W0, b0, W1, b1, W2, b2):
    n_nodes, dim = x.shape
    n_edges = edge_attr.shape[0]
    n_graphs = u.shape[0]

    src = edge_index[0].reshape(n_edges // _SCH, _SCH)
    sums, cnts = _make_sc_scatter(n_nodes, n_edges, dim)(edge_attr, src)

    out = pl.pallas_call(
        _tc_finish_body,
        out_shape=jax.ShapeDtypeStruct((n_graphs, dim), jnp.float32),
    )(sums, cnts, x, batch.reshape(1, n_nodes), u, W0, b0, W1, b1, W2, b2)
    return out


# speedup vs baseline: 8.9995x; 1.0030x over previous
import functools

import jax
import jax.numpy as jnp
from jax import lax
from jax.experimental import pallas as pl
from jax.experimental.pallas import tpu as pltpu
from jax.experimental.pallas import tpu_sc as plsc

_NC = 2
_NS = 16
_L = 16

_SCH = 80
_NSUB = 1
_GCH = _SCH * _NSUB


def _zero_fill_rows(ref, width):
    zero = jnp.zeros((_L,), jnp.float32)

    def _row(r, _):
        for j in range(width // _L):
            ref[r, pl.ds(j * _L, _L)] = zero
        return 0

    lax.fori_loop(0, ref.shape[0], _row, 0)


def _make_sc_scatter(n_nodes, n_edges, dim):
    mesh = plsc.VectorSubcoreMesh(core_axis_name="c", subcore_axis_name="s")

    @functools.partial(
        pl.kernel,
        out_type=(
            jax.ShapeDtypeStruct((_NC, n_nodes, dim), jnp.float32),
            jax.ShapeDtypeStruct((_NC, n_nodes, _L), jnp.float32),
        ),
        mesh=mesh,
        scratch_types=[
            pltpu.VMEM((2, _GCH, dim), jnp.float32),
            pltpu.VMEM((n_edges // (_NC * _NS) // _SCH, _SCH), jnp.int32),
            pltpu.VMEM((_SCH, _L), jnp.float32),
            pltpu.VMEM((_SCH, _L), jnp.float32),
            pltpu.VMEM_SHARED((n_nodes, dim), jnp.float32),
            pltpu.VMEM_SHARED((n_nodes, _L), jnp.float32),
            pltpu.SemaphoreType.DMA,
            pltpu.SemaphoreType.DMA,
            pltpu.SemaphoreType.DMA,
            pltpu.SemaphoreType.DMA,
        ],
        compiler_params=pltpu.CompilerParams(use_tc_tiling_on_sc=False),
    )
    def sc_scatter(edge_hbm, idx_hbm, sums_out, cnts_out, buf, idxbuf, onesb,
                   zc, acc, cacc, esem0, esem1, isem0, isem1):
        c = lax.axis_index("c")
        s = lax.axis_index("s")
        esem = (esem0, esem1)
        n_rows_idx = idxbuf.shape[0]

        wid = c * _NS + s
        pltpu.async_copy(idx_hbm.at[pl.ds(wid * n_rows_idx, n_rows_idx)],
                         idxbuf, isem0)

        one = jnp.ones((_L,), jnp.float32)

        def _ones_row(r, _):
            onesb[r] = one
            return 0

        lax.fori_loop(0, _SCH, _ones_row, 0)
        _zero_fill_rows(zc, _L)
        _zero_fill_rows(buf.at[0], dim)

        rows_pt = (n_nodes // _NS) // 8 * 8
        rem_rows = n_nodes - _NS * rows_pt
        row0 = s * rows_pt
        n_zfull = rows_pt // _GCH
        for j in range(n_zfull):
            pltpu.sync_copy(buf.at[0, pl.ds(0, _GCH)],
                            acc.at[pl.ds(row0 + j * _GCH, _GCH)])
        zrem_a = rows_pt - n_zfull * _GCH
        if zrem_a:
            pltpu.sync_copy(buf.at[0, pl.ds(0, zrem_a)],
                            acc.at[pl.ds(row0 + n_zfull * _GCH, zrem_a)])
        n_zfull_c = rows_pt // _SCH
        for j in range(n_zfull_c):
            pltpu.sync_copy(zc, cacc.at[pl.ds(row0 + j * _SCH, _SCH)])
        zrem = rows_pt - n_zfull_c * _SCH
        if zrem:
            pltpu.sync_copy(zc.at[pl.ds(0, zrem)],
                            cacc.at[pl.ds(row0 + n_zfull_c * _SCH, zrem)])

        @pl.when(s == _NS - 1)
        def _zero_tail():
            pltpu.sync_copy(buf.at[0, pl.ds(0, rem_rows)],
                            acc.at[pl.ds(_NS * rows_pt, rem_rows)])
            pltpu.sync_copy(zc.at[pl.ds(0, rem_rows)],
                            cacc.at[pl.ds(_NS * rows_pt, rem_rows)])

        pltpu.make_async_copy(idx_hbm.at[pl.ds(0, n_rows_idx)], idxbuf,
                              isem0).wait()
        plsc.subcore_barrier()

        edges_pw = n_edges // (_NC * _NS)
        wbase = wid * edges_pw
        n_iter = edges_pw // _GCH

        def _start_gather(i, b):
            pltpu.async_copy(edge_hbm.at[pl.ds(wbase + i * _GCH, _GCH)],
                             buf.at[b], esem[b])

        def _wait_gather(b):
            pltpu.make_async_copy(edge_hbm.at[pl.ds(0, _GCH)], buf.at[b],
                                  esem[b]).wait()

        def _consume(i, b):
            _wait_gather(b)
            for j in range(_NSUB):
                pltpu.sync_copy(buf.at[b, pl.ds(j * _SCH, _SCH)],
                                acc.at[idxbuf.at[i * _NSUB + j]], add=True)
                pltpu.sync_copy(onesb, cacc.at[idxbuf.at[i * _NSUB + j]],
                                add=True)

            @pl.when(i + 2 < n_iter)
            def _next():
                _start_gather(i + 2, b)

        _start_gather(0, 0)
        _start_gather(1, 1)

        def _outer(g, _):
            _consume(2 * g, 0)
            _consume(2 * g + 1, 1)
            return 0

        lax.fori_loop(0, n_iter // 2, _outer, 0)
        if n_iter % 2:
            _consume(n_iter - 1, 0)
        plsc.subcore_barrier()

        pltpu.sync_copy(acc.at[pl.ds(row0, rows_pt)],
                        sums_out.at[c, pl.ds(row0, rows_pt)])
        pltpu.sync_copy(cacc.at[pl.ds(row0, rows_pt)],
                        cnts_out.at[c, pl.ds(row0, rows_pt)])

        @pl.when(s == _NS - 1)
        def _copy_tail():
            pltpu.sync_copy(acc.at[pl.ds(_NS * rows_pt, rem_rows)],
                            sums_out.at[c, pl.ds(_NS * rows_pt, rem_rows)])
            pltpu.sync_copy(cacc.at[pl.ds(_NS * rows_pt, rem_rows)],
                            cnts_out.at[c, pl.ds(_NS * rows_pt, rem_rows)])

    return sc_scatter


def _tc_finish_body(sums_ref, cnts_ref, x_ref, batch_ref, u_ref, W0_ref,
                    b0_ref, W1_ref, b1_ref, W2_ref, b2_ref, out_ref):
    n_nodes = x_ref.shape[0]
    n_graphs = u_ref.shape[0]
    s = sums_ref[0] + sums_ref[1]
    cnt = jnp.sum(cnts_ref[0] + cnts_ref[1], axis=1, keepdims=True) * (1.0 / _L)
    ue_node = s / jnp.maximum(cnt, 1.0)
    bvec = batch_ref[:]
    giota = lax.broadcasted_iota(jnp.int32, (n_graphs, n_nodes), 0)
    onehot = (giota == bvec).astype(jnp.float32)
    acc_ue = jnp.dot(onehot, ue_node, preferred_element_type=jnp.float32)
    acc_uv = jnp.dot(onehot, x_ref[:], preferred_element_type=jnp.float32)
    npg = jnp.sum(onehot, axis=1, keepdims=True)
    inv = 1.0 / jnp.maximum(npg, 1.0)
    comb = jnp.concatenate([acc_ue * inv, acc_uv * inv, u_ref[:]], axis=1)
    h = jnp.maximum(
        jnp.dot(comb, W0_ref[:], preferred_element_type=jnp.float32)
        + b0_ref[:], 0.0)
    h = jnp.maximum(
        jnp.dot(h, W1_ref[:], preferred_element_type=jnp.float32)
        + b1_ref[:], 0.0)
    out_ref[:] = jnp.maximum(
        jnp.dot(h, W2_ref[:], preferred_element_type=jnp.float32)
        + b2_ref[:], 0.0)


def kernel(x, edge_index, edge_attr, u, batch, W0, b0, W1, b1, W2, b2):
    n_nodes, dim = x.shape
    n_edges = edge_attr.shape[0]
    n_graphs = u.shape[0]

    src = edge_index[0].reshape(n_edges // _SCH, _SCH)
    sums, cnts = _make_sc_scatter(n_nodes, n_edges, dim)(edge_attr, src)

    out = pl.pallas_call(
        _tc_finish_body,
        out_shape=jax.ShapeDtypeStruct((n_graphs, dim), jnp.float32),
    )(sums, cnts, x, batch.reshape(1, n_nodes), u, W0, b0, W1, b1, W2, b2)
    return out


# speedup vs baseline: 10.1642x; 1.1294x over previous
import functools

import jax
import jax.numpy as jnp
from jax import lax
from jax.experimental import pallas as pl
from jax.experimental.pallas import tpu as pltpu
from jax.experimental.pallas import tpu_sc as plsc

_NC = 2
_NS = 16
_L = 16

_SCH = 80
_NSUB = 1
_GCH = _SCH * _NSUB


def _zero_fill_rows(ref, width):
    zero = jnp.zeros((_L,), jnp.float32)

    def _row(r, _):
        for j in range(width // _L):
            ref[r, pl.ds(j * _L, _L)] = zero
        return 0

    lax.fori_loop(0, ref.shape[0], _row, 0)


def _make_sc_scatter(n_nodes, n_edges, dim):
    mesh = plsc.VectorSubcoreMesh(core_axis_name="c", subcore_axis_name="s")

    @functools.partial(
        pl.kernel,
        out_type=(
            jax.ShapeDtypeStruct((_NC, n_nodes, dim), jnp.float32),
            jax.ShapeDtypeStruct((_NC, n_nodes, _L), jnp.float32),
        ),
        mesh=mesh,
        scratch_types=[
            pltpu.VMEM((3, _GCH, dim), jnp.float32),
            pltpu.VMEM((3, _SCH), jnp.int32),
            pltpu.VMEM((_SCH, _L), jnp.float32),
            pltpu.VMEM((_SCH, _L), jnp.float32),
            pltpu.VMEM_SHARED((n_nodes, dim), jnp.float32),
            pltpu.VMEM_SHARED((n_nodes, _L), jnp.float32),
            pltpu.SemaphoreType.DMA((3,)),
            pltpu.SemaphoreType.DMA((3,)),
            pltpu.SemaphoreType.DMA((3,)),
            pltpu.SemaphoreType.DMA((3,)),
        ],
        compiler_params=pltpu.CompilerParams(use_tc_tiling_on_sc=False),
    )
    def sc_scatter(edge_hbm, idx_hbm, sums_out, cnts_out, buf, idxbuf, onesb,
                   zc, acc, cacc, esem, isem, ssem, osem):
        c = lax.axis_index("c")
        s = lax.axis_index("s")
        wid = c * _NS + s

        one = jnp.ones((_L,), jnp.float32)

        def _ones_row(r, _):
            onesb[r] = one
            return 0

        lax.fori_loop(0, _SCH, _ones_row, 0)
        _zero_fill_rows(zc, _L)
        _zero_fill_rows(buf.at[0], dim)

        rows_pt = (n_nodes // _NS) // 8 * 8
        rem_rows = n_nodes - _NS * rows_pt
        row0 = s * rows_pt
        n_zfull = rows_pt // _GCH
        for j in range(n_zfull):
            pltpu.sync_copy(buf.at[0, pl.ds(0, _GCH)],
                            acc.at[pl.ds(row0 + j * _GCH, _GCH)])
        zrem_a = rows_pt - n_zfull * _GCH
        if zrem_a:
            pltpu.sync_copy(buf.at[0, pl.ds(0, zrem_a)],
                            acc.at[pl.ds(row0 + n_zfull * _GCH, zrem_a)])
        n_zfull_c = rows_pt // _SCH
        for j in range(n_zfull_c):
            pltpu.sync_copy(zc, cacc.at[pl.ds(row0 + j * _SCH, _SCH)])
        zrem = rows_pt - n_zfull_c * _SCH
        if zrem:
            pltpu.sync_copy(zc.at[pl.ds(0, zrem)],
                            cacc.at[pl.ds(row0 + n_zfull_c * _SCH, zrem)])

        @pl.when(s == _NS - 1)
        def _zero_tail():
            pltpu.sync_copy(buf.at[0, pl.ds(0, rem_rows)],
                            acc.at[pl.ds(_NS * rows_pt, rem_rows)])
            pltpu.sync_copy(zc.at[pl.ds(0, rem_rows)],
                            cacc.at[pl.ds(_NS * rows_pt, rem_rows)])

        plsc.subcore_barrier()

        edges_pw = n_edges // (_NC * _NS)
        wbase = wid * edges_pw
        n_iter = edges_pw // _GCH
        idx_row0 = wbase // _SCH

        def _start_gather(i, b):
            pltpu.async_copy(edge_hbm.at[pl.ds(wbase + i * _GCH, _GCH)],
                             buf.at[b], esem.at[b])
            pltpu.async_copy(idx_hbm.at[idx_row0 + i], idxbuf.at[b],
                             isem.at[b])

        def _wait_gather(b):
            pltpu.make_async_copy(edge_hbm.at[pl.ds(0, _GCH)], buf.at[b],
                                  esem.at[b]).wait()
            pltpu.make_async_copy(idx_hbm.at[0], idxbuf.at[b],
                                  isem.at[b]).wait()

        def _wait_scatter(b):
            pltpu.make_async_copy(buf.at[b], acc.at[idxbuf.at[b]],
                                  ssem.at[b]).wait()
            pltpu.make_async_copy(onesb, cacc.at[idxbuf.at[b]],
                                  osem.at[b]).wait()

        def _consume(i, b, bprev):
            _wait_gather(b)
            pltpu.async_copy(buf.at[b], acc.at[idxbuf.at[b]], ssem.at[b],
                             add=True)
            pltpu.async_copy(onesb, cacc.at[idxbuf.at[b]], osem.at[b],
                             add=True)

            @pl.when(i + 2 < n_iter)
            def _next():
                @pl.when(i >= 1)
                def _w():
                    _wait_scatter(bprev)

                _start_gather(i + 2, (b + 2) % 3)

        _start_gather(0, 0)
        _start_gather(1, 1)

        def _outer(g, _):
            _consume(3 * g, 0, 2)
            _consume(3 * g + 1, 1, 0)
            _consume(3 * g + 2, 2, 1)
            return 0

        n_full = n_iter // 3
        lax.fori_loop(0, n_full, _outer, 0)
        for t in range(n_full * 3, n_iter):
            _consume(t, t % 3, (t - 1) % 3)

        _wait_scatter(0)
        _wait_scatter(1)
        _wait_scatter(2)
        plsc.subcore_barrier()

        pltpu.sync_copy(acc.at[pl.ds(row0, rows_pt)],
                        sums_out.at[c, pl.ds(row0, rows_pt)])
        pltpu.sync_copy(cacc.at[pl.ds(row0, rows_pt)],
                        cnts_out.at[c, pl.ds(row0, rows_pt)])

        @pl.when(s == _NS - 1)
        def _copy_tail():
            pltpu.sync_copy(acc.at[pl.ds(_NS * rows_pt, rem_rows)],
                            sums_out.at[c, pl.ds(_NS * rows_pt, rem_rows)])
            pltpu.sync_copy(cacc.at[pl.ds(_NS * rows_pt, rem_rows)],
                            cnts_out.at[c, pl.ds(_NS * rows_pt, rem_rows)])

    return sc_scatter


def _tc_finish_body(sums_ref, cnts_ref, x_ref, batch_ref, u_ref, W0_ref,
                    b0_ref, W1_ref, b1_ref, W2_ref, b2_ref, out_ref):
    n_nodes = x_ref.shape[0]
    n_graphs = u_ref.shape[0]
    s = sums_ref[0] + sums_ref[1]
    cnt = jnp.sum(cnts_ref[0] + cnts_ref[1], axis=1, keepdims=True) * (1.0 / _L)
    ue_node = s / jnp.maximum(cnt, 1.0)
    bvec = batch_ref[:]
    giota = lax.broadcasted_iota(jnp.int32, (n_graphs, n_nodes), 0)
    onehot = (giota == bvec).astype(jnp.float32)
    acc_ue = jnp.dot(onehot, ue_node, preferred_element_type=jnp.float32)
    acc_uv = jnp.dot(onehot, x_ref[:], preferred_element_type=jnp.float32)
    npg = jnp.sum(onehot, axis=1, keepdims=True)
    inv = 1.0 / jnp.maximum(npg, 1.0)
    comb = jnp.concatenate([acc_ue * inv, acc_uv * inv, u_ref[:]], axis=1)
    h = jnp.maximum(
        jnp.dot(comb, W0_ref[:], preferred_element_type=jnp.float32)
        + b0_ref[:], 0.0)
    h = jnp.maximum(
        jnp.dot(h, W1_ref[:], preferred_element_type=jnp.float32)
        + b1_ref[:], 0.0)
    out_ref[:] = jnp.maximum(
        jnp.dot(h, W2_ref[:], preferred_element_type=jnp.float32)
        + b2_ref[:], 0.0)


def kernel(x, edge_index, edge_attr, u, batch, W0, b0, W1, b1, W2, b2):
    n_nodes, dim = x.shape
    n_edges = edge_attr.shape[0]
    n_graphs = u.shape[0]

    src = edge_index[0].reshape(n_edges // _SCH, _SCH)
    sums, cnts = _make_sc_scatter(n_nodes, n_edges, dim)(edge_attr, src)

    out = pl.pallas_call(
        _tc_finish_body,
        out_shape=jax.ShapeDtypeStruct((n_graphs, dim), jnp.float32),
    )(sums, cnts, x, batch.reshape(1, n_nodes), u, W0, b0, W1, b1, W2, b2)
    return out


# speedup vs baseline: 10.2640x; 1.0098x over previous
import functools

import jax
import jax.numpy as jnp
from jax import lax
from jax.experimental import pallas as pl
from jax.experimental.pallas import tpu as pltpu
from jax.experimental.pallas import tpu_sc as plsc

_NC = 2
_NS = 16
_L = 16

_SCH = 80
_NSUB = 1
_GCH = _SCH * _NSUB


def _zero_fill_rows(ref, width):
    zero = jnp.zeros((_L,), jnp.float32)

    def _row(r, _):
        for j in range(width // _L):
            ref[r, pl.ds(j * _L, _L)] = zero
        return 0

    lax.fori_loop(0, ref.shape[0], _row, 0)


def _make_sc_scatter(n_nodes, n_edges, dim):
    mesh = plsc.VectorSubcoreMesh(core_axis_name="c", subcore_axis_name="s")

    @functools.partial(
        pl.kernel,
        out_type=(
            jax.ShapeDtypeStruct((_NC, n_nodes, dim), jnp.float32),
            jax.ShapeDtypeStruct((_NC, n_nodes, _L), jnp.float32),
        ),
        mesh=mesh,
        scratch_types=[
            pltpu.VMEM((3, _GCH, dim), jnp.float32),
            pltpu.VMEM((3, _SCH), jnp.int32),
            pltpu.VMEM((_SCH, _L), jnp.float32),
            pltpu.VMEM((_SCH, _L), jnp.float32),
            pltpu.VMEM_SHARED((n_nodes, dim), jnp.float32),
            pltpu.VMEM_SHARED((n_nodes, _L), jnp.float32),
            pltpu.SemaphoreType.DMA((3,)),
            pltpu.SemaphoreType.DMA((3,)),
            pltpu.SemaphoreType.DMA((3,)),
            pltpu.SemaphoreType.DMA((3,)),
        ],
        compiler_params=pltpu.CompilerParams(use_tc_tiling_on_sc=False),
    )
    def sc_scatter(edge_hbm, idx_hbm, sums_out, cnts_out, buf, idxbuf, onesb,
                   zc, acc, cacc, esem, isem, ssem, osem):
        c = lax.axis_index("c")
        s = lax.axis_index("s")
        wid = c * _NS + s

        one = jnp.ones((_L,), jnp.float32)

        def _ones_row(r, _):
            onesb[r] = one
            return 0

        lax.fori_loop(0, _SCH, _ones_row, 0)
        _zero_fill_rows(zc, _L)
        _zero_fill_rows(buf.at[0], dim)

        rows_pt = (n_nodes // _NS) // 8 * 8
        rem_rows = n_nodes - _NS * rows_pt
        row0 = s * rows_pt
        n_zfull = rows_pt // _GCH
        for j in range(n_zfull):
            pltpu.sync_copy(buf.at[0, pl.ds(0, _GCH)],
                            acc.at[pl.ds(row0 + j * _GCH, _GCH)])
        zrem_a = rows_pt - n_zfull * _GCH
        if zrem_a:
            pltpu.sync_copy(buf.at[0, pl.ds(0, zrem_a)],
                            acc.at[pl.ds(row0 + n_zfull * _GCH, zrem_a)])
        n_zfull_c = rows_pt // _SCH
        for j in range(n_zfull_c):
            pltpu.sync_copy(zc, cacc.at[pl.ds(row0 + j * _SCH, _SCH)])
        zrem = rows_pt - n_zfull_c * _SCH
        if zrem:
            pltpu.sync_copy(zc.at[pl.ds(0, zrem)],
                            cacc.at[pl.ds(row0 + n_zfull_c * _SCH, zrem)])

        @pl.when(s == _NS - 1)
        def _zero_tail():
            pltpu.sync_copy(buf.at[0, pl.ds(0, rem_rows)],
                            acc.at[pl.ds(_NS * rows_pt, rem_rows)])
            pltpu.sync_copy(zc.at[pl.ds(0, rem_rows)],
                            cacc.at[pl.ds(_NS * rows_pt, rem_rows)])

        plsc.subcore_barrier()

        edges_pw = n_edges // (_NC * _NS)
        wbase = wid * edges_pw
        n_iter = edges_pw // _GCH
        idx_row0 = wbase // _SCH

        def _start_gather(i, b):
            pltpu.async_copy(edge_hbm.at[pl.ds(wbase + i * _GCH, _GCH)],
                             buf.at[b], esem.at[b])
            pltpu.async_copy(idx_hbm.at[idx_row0 + i], idxbuf.at[b],
                             isem.at[b])

        def _wait_gather(b):
            pltpu.make_async_copy(edge_hbm.at[pl.ds(0, _GCH)], buf.at[b],
                                  esem.at[b]).wait()
            pltpu.make_async_copy(idx_hbm.at[0], idxbuf.at[b],
                                  isem.at[b]).wait()

        def _wait_scatter(b):
            pltpu.make_async_copy(buf.at[b], acc.at[idxbuf.at[b]],
                                  ssem.at[b]).wait()
            pltpu.make_async_copy(onesb, cacc.at[idxbuf.at[b]],
                                  osem.at[b]).wait()

        def _consume(i, b, bprev):
            _wait_gather(b)
            pltpu.async_copy(buf.at[b], acc.at[idxbuf.at[b]], ssem.at[b],
                             add=True)
            pltpu.async_copy(onesb, cacc.at[idxbuf.at[b]], osem.at[b],
                             add=True)

            @pl.when(i + 2 < n_iter)
            def _next():
                @pl.when(i >= 1)
                def _w():
                    _wait_scatter(bprev)

                _start_gather(i + 2, (b + 2) % 3)

        _start_gather(0, 0)
        _start_gather(1, 1)

        def _outer(g, _):
            _consume(3 * g, 0, 2)
            _consume(3 * g + 1, 1, 0)
            _consume(3 * g + 2, 2, 1)
            return 0

        n_full = n_iter // 3
        lax.fori_loop(0, n_full, _outer, 0)
        for t in range(n_full * 3, n_iter):
            _consume(t, t % 3, (t - 1) % 3)

        _wait_scatter(0)
        _wait_scatter(1)
        _wait_scatter(2)
        plsc.subcore_barrier()

        pltpu.sync_copy(acc.at[pl.ds(row0, rows_pt)],
                        sums_out.at[c, pl.ds(row0, rows_pt)])
        pltpu.sync_copy(cacc.at[pl.ds(row0, rows_pt)],
                        cnts_out.at[c, pl.ds(row0, rows_pt)])

        @pl.when(s == _NS - 1)
        def _copy_tail():
            pltpu.sync_copy(acc.at[pl.ds(_NS * rows_pt, rem_rows)],
                            sums_out.at[c, pl.ds(_NS * rows_pt, rem_rows)])
            pltpu.sync_copy(cacc.at[pl.ds(_NS * rows_pt, rem_rows)],
                            cnts_out.at[c, pl.ds(_NS * rows_pt, rem_rows)])

    return sc_scatter


def _tc_xpart_body(x_ref, batch_ref, uv_ref, npg_ref):
    n_nodes = x_ref.shape[0]
    n_graphs = uv_ref.shape[0]
    bvec = batch_ref[:]
    giota = lax.broadcasted_iota(jnp.int32, (n_graphs, n_nodes), 0)
    onehot = (giota == bvec).astype(jnp.float32)
    uv_ref[:] = jnp.dot(onehot, x_ref[:], preferred_element_type=jnp.float32)
    npg_ref[:] = jnp.broadcast_to(
        jnp.sum(onehot, axis=1, keepdims=True), npg_ref.shape)


def _tc_finish_body(sums_ref, cnts_ref, batch_ref, u_ref, uv_ref, npg_ref,
                    W0_ref, b0_ref, W1_ref, b1_ref, W2_ref, b2_ref, out_ref):
    n_nodes = sums_ref.shape[1]
    n_graphs = u_ref.shape[0]
    s = sums_ref[0] + sums_ref[1]
    cnt = jnp.sum(cnts_ref[0] + cnts_ref[1], axis=1, keepdims=True) * (1.0 / _L)
    ue_node = s / jnp.maximum(cnt, 1.0)
    bvec = batch_ref[:]
    giota = lax.broadcasted_iota(jnp.int32, (n_graphs, n_nodes), 0)
    onehot = (giota == bvec).astype(jnp.float32)
    acc_ue = jnp.dot(onehot, ue_node, preferred_element_type=jnp.float32)
    inv = 1.0 / jnp.maximum(npg_ref[:], 1.0)
    comb = jnp.concatenate([acc_ue * inv, uv_ref[:] * inv, u_ref[:]], axis=1)
    h = jnp.maximum(
        jnp.dot(comb, W0_ref[:], preferred_element_type=jnp.float32)
        + b0_ref[:], 0.0)
    h = jnp.maximum(
        jnp.dot(h, W1_ref[:], preferred_element_type=jnp.float32)
        + b1_ref[:], 0.0)
    out_ref[:] = jnp.maximum(
        jnp.dot(h, W2_ref[:], preferred_element_type=jnp.float32)
        + b2_ref[:], 0.0)


def kernel(x, edge_index, edge_attr, u, batch, W0, b0, W1, b1, W2, b2):
    n_nodes, dim = x.shape
    n_edges = edge_attr.shape[0]
    n_graphs = u.shape[0]

    src = edge_index[0].reshape(n_edges // _SCH, _SCH)
    sums, cnts = _make_sc_scatter(n_nodes, n_edges, dim)(edge_attr, src)

    batch2 = batch.reshape(1, n_nodes)
    uv, npg = pl.pallas_call(
        _tc_xpart_body,
        out_shape=(jax.ShapeDtypeStruct((n_graphs, dim), jnp.float32),
                   jax.ShapeDtypeStruct((n_graphs, dim), jnp.float32)),
    )(x, batch2)

    out = pl.pallas_call(
        _tc_finish_body,
        out_shape=jax.ShapeDtypeStruct((n_graphs, dim), jnp.float32),
    )(sums, cnts, batch2, u, uv, npg, W0, b0, W1, b1, W2, b2)
    return out


# speedup vs baseline: 10.4701x; 1.0201x over previous
import functools

import jax
import jax.numpy as jnp
from jax import lax
from jax.experimental import pallas as pl
from jax.experimental.pallas import tpu as pltpu
from jax.experimental.pallas import tpu_sc as plsc

_NC = 2
_NS = 16
_L = 16

_SCH = 80
_NSUB = 1
_GCH = _SCH * _NSUB


def _zero_fill_rows(ref, width):
    zero = jnp.zeros((_L,), jnp.float32)

    def _row(r, _):
        for j in range(width // _L):
            ref[r, pl.ds(j * _L, _L)] = zero
        return 0

    lax.fori_loop(0, ref.shape[0], _row, 0)


def _make_sc_scatter(n_nodes, n_edges, dim):
    mesh = plsc.VectorSubcoreMesh(core_axis_name="c", subcore_axis_name="s")

    @functools.partial(
        pl.kernel,
        out_type=(
            jax.ShapeDtypeStruct((_NC, n_nodes, dim), jnp.float32),
            jax.ShapeDtypeStruct((_NC, n_nodes, _L), jnp.float32),
        ),
        mesh=mesh,
        scratch_types=[
            pltpu.VMEM((3, _GCH, dim), jnp.float32),
            pltpu.VMEM((3, _SCH), jnp.int32),
            pltpu.VMEM((_SCH, _L), jnp.float32),
            pltpu.VMEM((_SCH, _L), jnp.float32),
            pltpu.VMEM_SHARED((n_nodes, dim), jnp.float32),
            pltpu.VMEM_SHARED((n_nodes, _L), jnp.float32),
            pltpu.SemaphoreType.DMA((3,)),
            pltpu.SemaphoreType.DMA((3,)),
            pltpu.SemaphoreType.DMA((3,)),
            pltpu.SemaphoreType.DMA((3,)),
        ],
        compiler_params=pltpu.CompilerParams(use_tc_tiling_on_sc=False),
    )
    def sc_scatter(edge_hbm, idx_hbm, sums_out, cnts_out, buf, idxbuf, onesb,
                   zc, acc, cacc, esem, isem, ssem, osem):
        c = lax.axis_index("c")
        s = lax.axis_index("s")
        wid = c * _NS + s

        one = jnp.ones((_L,), jnp.float32)

        def _ones_row(r, _):
            onesb[r] = one
            return 0

        edges_pw0 = n_edges // (_NC * _NS)
        wbase0 = wid * edges_pw0
        pltpu.async_copy(edge_hbm.at[pl.ds(wbase0, _GCH)], buf.at[0],
                         esem.at[0])
        pltpu.async_copy(idx_hbm.at[wbase0 // _SCH], idxbuf.at[0], isem.at[0])
        pltpu.async_copy(edge_hbm.at[pl.ds(wbase0 + _GCH, _GCH)], buf.at[1],
                         esem.at[1])
        pltpu.async_copy(idx_hbm.at[wbase0 // _SCH + 1], idxbuf.at[1],
                         isem.at[1])

        lax.fori_loop(0, _SCH, _ones_row, 0)
        _zero_fill_rows(zc, _L)
        _zero_fill_rows(buf.at[2], dim)

        rows_pt = (n_nodes // _NS) // 8 * 8
        rem_rows = n_nodes - _NS * rows_pt
        row0 = s * rows_pt
        zcopies = []
        n_zfull = rows_pt // _GCH
        for j in range(n_zfull):
            zcopies.append((buf.at[2, pl.ds(0, _GCH)],
                            acc.at[pl.ds(row0 + j * _GCH, _GCH)]))
        zrem_a = rows_pt - n_zfull * _GCH
        if zrem_a:
            zcopies.append((buf.at[2, pl.ds(0, zrem_a)],
                            acc.at[pl.ds(row0 + n_zfull * _GCH, zrem_a)]))
        n_zfull_c = rows_pt // _SCH
        for j in range(n_zfull_c):
            zcopies.append((zc, cacc.at[pl.ds(row0 + j * _SCH, _SCH)]))
        zrem = rows_pt - n_zfull_c * _SCH
        if zrem:
            zcopies.append((zc.at[pl.ds(0, zrem)],
                            cacc.at[pl.ds(row0 + n_zfull_c * _SCH, zrem)]))
        for src_r, dst_r in zcopies:
            pltpu.async_copy(src_r, dst_r, ssem.at[0])

        @pl.when(s == _NS - 1)
        def _zero_tail():
            pltpu.sync_copy(buf.at[2, pl.ds(0, rem_rows)],
                            acc.at[pl.ds(_NS * rows_pt, rem_rows)])
            pltpu.sync_copy(zc.at[pl.ds(0, rem_rows)],
                            cacc.at[pl.ds(_NS * rows_pt, rem_rows)])

        for src_r, dst_r in zcopies:
            pltpu.make_async_copy(src_r, dst_r, ssem.at[0]).wait()
        plsc.subcore_barrier()

        edges_pw = n_edges // (_NC * _NS)
        wbase = wid * edges_pw
        n_iter = edges_pw // _GCH
        idx_row0 = wbase // _SCH

        def _start_gather(i, b):
            pltpu.async_copy(edge_hbm.at[pl.ds(wbase + i * _GCH, _GCH)],
                             buf.at[b], esem.at[b])
            pltpu.async_copy(idx_hbm.at[idx_row0 + i], idxbuf.at[b],
                             isem.at[b])

        def _wait_gather(b):
            pltpu.make_async_copy(edge_hbm.at[pl.ds(0, _GCH)], buf.at[b],
                                  esem.at[b]).wait()
            pltpu.make_async_copy(idx_hbm.at[0], idxbuf.at[b],
                                  isem.at[b]).wait()

        def _wait_scatter(b):
            pltpu.make_async_copy(buf.at[b], acc.at[idxbuf.at[b]],
                                  ssem.at[b]).wait()
            pltpu.make_async_copy(onesb, cacc.at[idxbuf.at[b]],
                                  osem.at[b]).wait()

        def _consume(i, b, bprev):
            _wait_gather(b)
            pltpu.async_copy(buf.at[b], acc.at[idxbuf.at[b]], ssem.at[b],
                             add=True)
            pltpu.async_copy(onesb, cacc.at[idxbuf.at[b]], osem.at[b],
                             add=True)

            @pl.when(i + 2 < n_iter)
            def _next():
                @pl.when(i >= 1)
                def _w():
                    _wait_scatter(bprev)

                _start_gather(i + 2, (b + 2) % 3)

        def _outer(g, _):
            _consume(3 * g, 0, 2)
            _consume(3 * g + 1, 1, 0)
            _consume(3 * g + 2, 2, 1)
            return 0

        n_full = n_iter // 3
        lax.fori_loop(0, n_full, _outer, 0)
        for t in range(n_full * 3, n_iter):
            _consume(t, t % 3, (t - 1) % 3)

        _wait_scatter(0)
        _wait_scatter(1)
        _wait_scatter(2)
        plsc.subcore_barrier()

        pltpu.sync_copy(acc.at[pl.ds(row0, rows_pt)],
                        sums_out.at[c, pl.ds(row0, rows_pt)])
        pltpu.sync_copy(cacc.at[pl.ds(row0, rows_pt)],
                        cnts_out.at[c, pl.ds(row0, rows_pt)])

        @pl.when(s == _NS - 1)
        def _copy_tail():
            pltpu.sync_copy(acc.at[pl.ds(_NS * rows_pt, rem_rows)],
                            sums_out.at[c, pl.ds(_NS * rows_pt, rem_rows)])
            pltpu.sync_copy(cacc.at[pl.ds(_NS * rows_pt, rem_rows)],
                            cnts_out.at[c, pl.ds(_NS * rows_pt, rem_rows)])

    return sc_scatter


def _tc_xpart_body(x_ref, batch_ref, uv_ref, npg_ref):
    n_nodes = x_ref.shape[0]
    n_graphs = uv_ref.shape[0]
    bvec = batch_ref[:]
    giota = lax.broadcasted_iota(jnp.int32, (n_graphs, n_nodes), 0)
    onehot = (giota == bvec).astype(jnp.float32)
    uv_ref[:] = jnp.dot(onehot, x_ref[:], preferred_element_type=jnp.float32)
    npg_ref[:] = jnp.broadcast_to(
        jnp.sum(onehot, axis=1, keepdims=True), npg_ref.shape)


def _tc_finish_body(sums_ref, cnts_ref, batch_ref, u_ref, uv_ref, npg_ref,
                    W0_ref, b0_ref, W1_ref, b1_ref, W2_ref, b2_ref, out_ref):
    n_nodes = sums_ref.shape[1]
    n_graphs = u_ref.shape[0]
    s = sums_ref[0] + sums_ref[1]
    cnt = jnp.sum(cnts_ref[0] + cnts_ref[1], axis=1, keepdims=True) * (1.0 / _L)
    ue_node = s / jnp.maximum(cnt, 1.0)
    bvec = batch_ref[:]
    giota = lax.broadcasted_iota(jnp.int32, (n_graphs, n_nodes), 0)
    onehot = (giota == bvec).astype(jnp.float32)
    acc_ue = jnp.dot(onehot, ue_node, preferred_element_type=jnp.float32)
    inv = 1.0 / jnp.maximum(npg_ref[:], 1.0)
    comb = jnp.concatenate([acc_ue * inv, uv_ref[:] * inv, u_ref[:]], axis=1)
    h = jnp.maximum(
        jnp.dot(comb, W0_ref[:], preferred_element_type=jnp.float32)
        + b0_ref[:], 0.0)
    h = jnp.maximum(
        jnp.dot(h, W1_ref[:], preferred_element_type=jnp.float32)
        + b1_ref[:], 0.0)
    out_ref[:] = jnp.maximum(
        jnp.dot(h, W2_ref[:], preferred_element_type=jnp.float32)
        + b2_ref[:], 0.0)


def kernel(x, edge_index, edge_attr, u, batch, W0, b0, W1, b1, W2, b2):
    n_nodes, dim = x.shape
    n_edges = edge_attr.shape[0]
    n_graphs = u.shape[0]

    src = edge_index[0].reshape(n_edges // _SCH, _SCH)
    sums, cnts = _make_sc_scatter(n_nodes, n_edges, dim)(edge_attr, src)

    batch2 = batch.reshape(1, n_nodes)
    uv, npg = pl.pallas_call(
        _tc_xpart_body,
        out_shape=(jax.ShapeDtypeStruct((n_graphs, dim), jnp.float32),
                   jax.ShapeDtypeStruct((n_graphs, dim), jnp.float32)),
    )(x, batch2)

    out = pl.pallas_call(
        _tc_finish_body,
        out_shape=jax.ShapeDtypeStruct((n_graphs, dim), jnp.float32),
    )(sums, cnts, batch2, u, uv, npg, W0, b0, W1, b1, W2, b2)
    return out
